# baseline (speedup 1.0000x reference)
import jax
import jax.numpy as jnp
from jax.experimental import pallas as pl

N_ROWS = 16384
N_COLS = 1024
BM = 3712


def _fma_kernel(x_ref, mu_ref, sigma_ref, o_ref):
    o_ref[...] = mu_ref[...] + x_ref[...] * sigma_ref[...]


def kernel(x, mu, sigma):
    mu2 = mu.reshape(1, N_COLS)
    sigma2 = sigma.reshape(1, N_COLS)
    return pl.pallas_call(
        _fma_kernel,
        grid=(pl.cdiv(N_ROWS, BM),),
        in_specs=[
            pl.BlockSpec((BM, N_COLS), lambda i: (i, 0)),
            pl.BlockSpec((1, N_COLS), lambda i: (0, 0)),
            pl.BlockSpec((1, N_COLS), lambda i: (0, 0)),
        ],
        out_specs=pl.BlockSpec((BM, N_COLS), lambda i: (i, 0)),
        out_shape=jax.ShapeDtypeStruct((N_ROWS, N_COLS), x.dtype),
    )(x, mu2, sigma2)
